# Initial kernel scaffold; baseline (speedup 1.0000x reference)
import jax
import jax.numpy as jnp
from jax.experimental import pallas as pl


def kernel(indices_x, indices_y, table):
    raise NotImplementedError("write your pallas kernel here")


# speedup vs baseline: 4.2194x; 4.2194x over previous
import functools

import jax
import jax.numpy as jnp
from jax import lax
from jax.experimental import pallas as pl
from jax.experimental.pallas import tpu as pltpu
from jax.experimental.pallas import tpu_sc as plsc

DIM = 64
BATCH = 4096
SEQLEN = 200

NC = 2
NS = 16
NW = NC * NS

N = 2 * BATCH * SEQLEN
R = N // NW
C = 512
G = C // 128
N_CHUNKS = R // C


def _gather_rows(idx2d, table):
    mesh = plsc.VectorSubcoreMesh(core_axis_name="c", subcore_axis_name="s")

    @functools.partial(
        pl.kernel,
        out_type=jax.ShapeDtypeStruct((N, DIM), jnp.float32),
        mesh=mesh,
        compiler_params=pltpu.CompilerParams(use_tc_tiling_on_sc=False),
        scratch_types=[
            pltpu.VMEM((G, 128), jnp.int32),
            pltpu.VMEM((C, DIM), jnp.float32),
            pltpu.SemaphoreType.DMA,
        ],
    )
    def k(idx_hbm, table_hbm, out_hbm, idx_v, rows_v, gsem):
        wid = lax.axis_index("s") * NC + lax.axis_index("c")
        irow0 = wid * (R // 128)
        out0 = wid * R

        @pl.loop(0, N_CHUNKS)
        def _chunk(g):
            pltpu.sync_copy(idx_hbm.at[pl.ds(irow0 + g * G, G)], idx_v)
            copies = [
                pltpu.async_copy(
                    table_hbm.at[idx_v.at[j]],
                    rows_v.at[pl.ds(j * 128, 128)],
                    gsem,
                )
                for j in range(G)
            ]
            for cp in copies:
                cp.wait()
            pltpu.sync_copy(rows_v, out_hbm.at[pl.ds(out0 + g * C, C)])

    return k(idx2d, table)


def kernel(indices_x, indices_y, table):
    flat = jnp.concatenate([indices_x.reshape(-1), indices_y.reshape(-1)])
    out = _gather_rows(flat.reshape(-1, 128), table)
    return out.reshape(2, BATCH, SEQLEN, DIM)
